# Initial kernel scaffold; baseline (speedup 1.0000x reference)
import jax
import jax.numpy as jnp
from jax.experimental import pallas as pl


def kernel(x, edge_index, edge_attr, W1, b1, W2, b2, W3, b3, U1, c1, U2, c2, U3, c3):
    raise NotImplementedError("write your pallas kernel here")


# speedup vs baseline: 1.4647x; 1.4647x over previous
import functools

import jax
import jax.numpy as jnp
from jax import lax
from jax.experimental import pallas as pl
from jax.experimental.pallas import tpu as pltpu
from jax.experimental.pallas import tpu_sc as plsc

_DN = (((1,), (1,)), ((), ()))


def _xw_matmul(x, W1x):
    N, D = x.shape
    H = W1x.shape[0]
    BN = 2000

    def body(x_ref, w_ref, o_ref):
        o_ref[...] = lax.dot_general(x_ref[...], w_ref[...], _DN,
                                     preferred_element_type=jnp.float32)

    return pl.pallas_call(
        body,
        grid=(N // BN,),
        in_specs=[pl.BlockSpec((BN, D), lambda i: (i, 0)),
                  pl.BlockSpec((H, D), lambda i: (0, 0))],
        out_specs=pl.BlockSpec((BN, H), lambda i: (i, 0)),
        out_shape=jax.ShapeDtypeStruct((N, H), jnp.float32),
    )(x, W1x)


def _sc_gather(table, idx):
    N, H = table.shape
    E = idx.shape[0]
    NC, NS = 2, 16
    NW = NC * NS
    epw = E // NW
    CH = 80
    nch = epw // CH
    mesh = plsc.VectorSubcoreMesh(core_axis_name="c", subcore_axis_name="s")

    @functools.partial(
        pl.kernel,
        out_type=jax.ShapeDtypeStruct((E, H), jnp.float32),
        mesh=mesh,
        compiler_params=pltpu.CompilerParams(use_tc_tiling_on_sc=False),
        scratch_types=[
            pltpu.VMEM((CH,), jnp.int32),
            pltpu.VMEM((CH, H), jnp.float32),
            pltpu.SemaphoreType.DMA,
        ],
    )
    def k(table_hbm, idx_hbm, out_hbm, idx_v, rows_v, sem):
        wid = lax.axis_index("s") * NC + lax.axis_index("c")
        base = wid * epw

        @pl.loop(0, nch)
        def _(i):
            off = base + i * CH
            pltpu.sync_copy(idx_hbm.at[pl.ds(off, CH)], idx_v)
            pltpu.async_copy(table_hbm.at[idx_v], rows_v, sem).wait()
            pltpu.sync_copy(rows_v, out_hbm.at[pl.ds(off, CH)])

    return k(table, idx)


def _edge_mlp(xwg, ea, W1e, b1, W2, b2, W3, b3):
    E, H = xwg.shape
    DE = ea.shape[1]
    BE = 2000

    def body(xwg_ref, ea_ref, w1e_ref, b1_ref, w2_ref, b2_ref, w3_ref,
             b3_ref, o_ref):
        h1 = xwg_ref[...] + lax.dot_general(
            ea_ref[...], w1e_ref[...], _DN,
            preferred_element_type=jnp.float32) + b1_ref[...]
        h1 = jnp.maximum(h1, 0.0)
        h2 = jnp.maximum(
            lax.dot_general(h1, w2_ref[...], _DN,
                            preferred_element_type=jnp.float32) + b2_ref[...],
            0.0)
        o_ref[...] = lax.dot_general(
            h2, w3_ref[...], _DN,
            preferred_element_type=jnp.float32) + b3_ref[...]

    full = lambda shape: pl.BlockSpec(shape, lambda i: (0,) * len(shape))
    return pl.pallas_call(
        body,
        grid=(E // BE,),
        in_specs=[pl.BlockSpec((BE, H), lambda i: (i, 0)),
                  pl.BlockSpec((BE, DE), lambda i: (i, 0)),
                  full((H, DE)), full((1, H)),
                  full((H, H)), full((1, H)),
                  full((H, H)), full((1, H))],
        out_specs=pl.BlockSpec((BE, H), lambda i: (i, 0)),
        out_shape=jax.ShapeDtypeStruct((E, H), jnp.float32),
    )(xwg, ea, W1e, b1, W2, b2, W3, b3)


def _sc_scatter_add(m, dst, zeros):
    E, H = m.shape
    N = zeros.shape[0]
    NC, NS = 2, 16
    NW = NC * NS
    epw = E // NW
    CH = 80
    nch = epw // CH
    rpw = N // NS
    mesh = plsc.VectorSubcoreMesh(core_axis_name="c", subcore_axis_name="s")

    @functools.partial(
        pl.kernel,
        out_type=jax.ShapeDtypeStruct((NC, N, H), jnp.float32),
        mesh=mesh,
        compiler_params=pltpu.CompilerParams(use_tc_tiling_on_sc=False),
        scratch_types=[
            pltpu.VMEM((CH,), jnp.int32),
            pltpu.VMEM((CH, H), jnp.float32),
            pltpu.VMEM_SHARED((N, H), jnp.float32),
        ],
    )
    def k(m_hbm, dst_hbm, zeros_hbm, z_hbm, idx_v, rows_v, acc_sh):
        c = lax.axis_index("c")
        s = lax.axis_index("s")
        wid = s * NC + c
        r0 = s * rpw
        pltpu.sync_copy(zeros_hbm.at[pl.ds(r0, rpw)],
                        acc_sh.at[pl.ds(r0, rpw)])
        plsc.subcore_barrier()

        base = wid * epw

        @pl.loop(0, nch)
        def _(i):
            off = base + i * CH
            pltpu.sync_copy(dst_hbm.at[pl.ds(off, CH)], idx_v)
            pltpu.sync_copy(m_hbm.at[pl.ds(off, CH)], rows_v)
            pltpu.sync_copy(rows_v, acc_sh.at[idx_v], add=True)

        plsc.subcore_barrier()
        pltpu.sync_copy(acc_sh.at[pl.ds(r0, rpw)],
                        z_hbm.at[c, pl.ds(r0, rpw)])

    return k(m, dst, zeros)


def _node_mlp(x, zp, U1x, U1z, c1, U2, c2, U3, c3):
    N, D = x.shape
    H = U1z.shape[1]
    OUT = U3.shape[0]
    BN = 2000

    def body(x_ref, zp_ref, u1x_ref, u1z_ref, c1_ref, u2_ref, c2_ref,
             u3_ref, c3_ref, o_ref):
        z = zp_ref[0] + zp_ref[1]
        t1 = jnp.maximum(
            lax.dot_general(x_ref[...], u1x_ref[...], _DN,
                            preferred_element_type=jnp.float32)
            + lax.dot_general(z, u1z_ref[...], _DN,
                              preferred_element_type=jnp.float32)
            + c1_ref[...], 0.0)
        t2 = jnp.maximum(
            lax.dot_general(t1, u2_ref[...], _DN,
                            preferred_element_type=jnp.float32) + c2_ref[...],
            0.0)
        o_ref[...] = lax.dot_general(
            t2, u3_ref[...], _DN,
            preferred_element_type=jnp.float32) + c3_ref[...]

    full = lambda shape: pl.BlockSpec(shape, lambda i: (0,) * len(shape))
    return pl.pallas_call(
        body,
        grid=(N // BN,),
        in_specs=[pl.BlockSpec((BN, D), lambda i: (i, 0)),
                  pl.BlockSpec((2, BN, H), lambda i: (0, i, 0)),
                  full((H, D)), full((H, H)), full((1, H)),
                  full((H, H)), full((1, H)),
                  full((OUT, H)), full((1, OUT))],
        out_specs=pl.BlockSpec((BN, OUT), lambda i: (i, 0)),
        out_shape=jax.ShapeDtypeStruct((N, OUT), jnp.float32),
    )(x, zp, U1x, U1z, c1, U2, c2, U3, c3)


def kernel(x, edge_index, edge_attr, W1, b1, W2, b2, W3, b3,
           U1, c1, U2, c2, U3, c3):
    N, D = x.shape
    H = W1.shape[0]
    src = edge_index[0]
    dst = edge_index[1]
    W1x, W1e = W1[:, :D], W1[:, D:]
    U1x, U1z = U1[:, :D], U1[:, D:]

    xw = _xw_matmul(x, W1x)
    xwg = _sc_gather(xw, src)
    m = _edge_mlp(xwg, edge_attr, W1e, b1.reshape(1, -1), W2,
                  b2.reshape(1, -1), W3, b3.reshape(1, -1))
    zp = _sc_scatter_add(m, dst, jnp.zeros((N, H), jnp.float32))
    h = _node_mlp(x, zp, U1x, U1z, c1.reshape(1, -1), U2,
                  c2.reshape(1, -1), U3, c3.reshape(1, -1))
    return h


# speedup vs baseline: 1.7644x; 1.2046x over previous
import functools

import jax
import jax.numpy as jnp
from jax import lax
from jax.experimental import pallas as pl
from jax.experimental.pallas import tpu as pltpu
from jax.experimental.pallas import tpu_sc as plsc

_DN = (((1,), (1,)), ((), ()))


def _xw_matmul(x, W1x):
    N, D = x.shape
    H = W1x.shape[0]
    BN = 2000

    def body(x_ref, w_ref, o_ref):
        o_ref[...] = lax.dot_general(x_ref[...], w_ref[...], _DN,
                                     preferred_element_type=jnp.float32)

    return pl.pallas_call(
        body,
        grid=(N // BN,),
        in_specs=[pl.BlockSpec((BN, D), lambda i: (i, 0)),
                  pl.BlockSpec((H, D), lambda i: (0, 0))],
        out_specs=pl.BlockSpec((BN, H), lambda i: (i, 0)),
        out_shape=jax.ShapeDtypeStruct((N, H), jnp.float32),
    )(x, W1x)


_NBUF = 5


def _sc_gather(table, idx):
    N, H = table.shape
    E = idx.shape[0]
    NC, NS = 2, 16
    NW = NC * NS
    epw = E // NW
    CH = 80
    nch = epw // CH
    assert nch % _NBUF == 0
    mesh = plsc.VectorSubcoreMesh(core_axis_name="c", subcore_axis_name="s")

    @functools.partial(
        pl.kernel,
        out_type=jax.ShapeDtypeStruct((E, H), jnp.float32),
        mesh=mesh,
        compiler_params=pltpu.CompilerParams(use_tc_tiling_on_sc=False),
        scratch_types=[
            pltpu.VMEM((epw,), jnp.int32),
            [pltpu.VMEM((CH, H), jnp.float32) for _ in range(_NBUF)],
            [pltpu.SemaphoreType.DMA for _ in range(_NBUF)],
        ],
    )
    def k(table_hbm, idx_hbm, out_hbm, idx_v, rows_bufs, sems):
        wid = lax.axis_index("s") * NC + lax.axis_index("c")
        base = wid * epw
        pltpu.sync_copy(idx_hbm.at[pl.ds(base, epw)], idx_v)
        for b in range(_NBUF):
            pltpu.async_copy(table_hbm.at[idx_v.at[pl.ds(b * CH, CH)]],
                             rows_bufs[b], sems[b])

        @pl.loop(0, nch, step=_NBUF)
        def _(c0):
            for b in range(_NBUF):
                c = c0 + b
                pltpu.make_async_copy(
                    table_hbm.at[idx_v.at[pl.ds(c * CH, CH)]],
                    rows_bufs[b], sems[b]).wait()
                pltpu.sync_copy(rows_bufs[b],
                                out_hbm.at[pl.ds(base + c * CH, CH)])
                cn = c + _NBUF

                @pl.when(cn < nch)
                def _():
                    pltpu.async_copy(
                        table_hbm.at[idx_v.at[pl.ds(cn * CH, CH)]],
                        rows_bufs[b], sems[b])

    return k(table, idx)


def _edge_mlp(xwg, ea, W1e, b1, W2, b2, W3, b3):
    E, H = xwg.shape
    DE = ea.shape[1]
    BE = 2000

    def body(xwg_ref, ea_ref, w1e_ref, b1_ref, w2_ref, b2_ref, w3_ref,
             b3_ref, o_ref):
        h1 = xwg_ref[...] + lax.dot_general(
            ea_ref[...], w1e_ref[...], _DN,
            preferred_element_type=jnp.float32) + b1_ref[...]
        h1 = jnp.maximum(h1, 0.0)
        h2 = jnp.maximum(
            lax.dot_general(h1, w2_ref[...], _DN,
                            preferred_element_type=jnp.float32) + b2_ref[...],
            0.0)
        o_ref[...] = lax.dot_general(
            h2, w3_ref[...], _DN,
            preferred_element_type=jnp.float32) + b3_ref[...]

    full = lambda shape: pl.BlockSpec(shape, lambda i: (0,) * len(shape))
    return pl.pallas_call(
        body,
        grid=(E // BE,),
        in_specs=[pl.BlockSpec((BE, H), lambda i: (i, 0)),
                  pl.BlockSpec((BE, DE), lambda i: (i, 0)),
                  full((H, DE)), full((1, H)),
                  full((H, H)), full((1, H)),
                  full((H, H)), full((1, H))],
        out_specs=pl.BlockSpec((BE, H), lambda i: (i, 0)),
        out_shape=jax.ShapeDtypeStruct((E, H), jnp.float32),
    )(xwg, ea, W1e, b1, W2, b2, W3, b3)


def _sc_scatter_add(m, dst, zeros):
    E, H = m.shape
    N = zeros.shape[0]
    NC, NS = 2, 16
    NW = NC * NS
    epw = E // NW
    CH = 40
    nch = epw // CH
    rpw = N // NS
    mesh = plsc.VectorSubcoreMesh(core_axis_name="c", subcore_axis_name="s")

    assert nch % _NBUF == 0

    @functools.partial(
        pl.kernel,
        out_type=jax.ShapeDtypeStruct((NC, N, H), jnp.float32),
        mesh=mesh,
        compiler_params=pltpu.CompilerParams(use_tc_tiling_on_sc=False),
        scratch_types=[
            [pltpu.VMEM((CH,), jnp.int32) for _ in range(_NBUF)],
            [pltpu.VMEM((CH, H), jnp.float32) for _ in range(_NBUF)],
            pltpu.VMEM_SHARED((N, H), jnp.float32),
            [pltpu.SemaphoreType.DMA for _ in range(_NBUF)],
            [pltpu.SemaphoreType.DMA for _ in range(_NBUF)],
        ],
    )
    def k(m_hbm, dst_hbm, zeros_hbm, z_hbm, idx_bufs, rows_bufs, acc_sh,
          isems, rsems):
        c = lax.axis_index("c")
        s = lax.axis_index("s")
        wid = s * NC + c
        r0 = s * rpw
        pltpu.sync_copy(zeros_hbm.at[pl.ds(r0, rpw)],
                        acc_sh.at[pl.ds(r0, rpw)])
        plsc.subcore_barrier()

        base = wid * epw
        for b in range(_NBUF):
            off = base + b * CH
            pltpu.async_copy(dst_hbm.at[pl.ds(off, CH)], idx_bufs[b],
                             isems[b])
            pltpu.async_copy(m_hbm.at[pl.ds(off, CH)], rows_bufs[b],
                             rsems[b])

        @pl.loop(0, nch, step=_NBUF)
        def _(c0):
            for b in range(_NBUF):
                ci = c0 + b
                off = base + ci * CH
                pltpu.make_async_copy(dst_hbm.at[pl.ds(off, CH)],
                                      idx_bufs[b], isems[b]).wait()
                pltpu.make_async_copy(m_hbm.at[pl.ds(off, CH)],
                                      rows_bufs[b], rsems[b]).wait()
                pltpu.sync_copy(rows_bufs[b], acc_sh.at[idx_bufs[b]],
                                add=True)
                cn = ci + _NBUF

                @pl.when(cn < nch)
                def _():
                    offn = base + cn * CH
                    pltpu.async_copy(dst_hbm.at[pl.ds(offn, CH)],
                                     idx_bufs[b], isems[b])
                    pltpu.async_copy(m_hbm.at[pl.ds(offn, CH)],
                                     rows_bufs[b], rsems[b])

        plsc.subcore_barrier()
        pltpu.sync_copy(acc_sh.at[pl.ds(r0, rpw)],
                        z_hbm.at[c, pl.ds(r0, rpw)])

    return k(m, dst, zeros)


def _node_mlp(x, zp, U1x, U1z, c1, U2, c2, U3, c3):
    N, D = x.shape
    H = U1z.shape[1]
    OUT = U3.shape[0]
    BN = 2000

    def body(x_ref, zp_ref, u1x_ref, u1z_ref, c1_ref, u2_ref, c2_ref,
             u3_ref, c3_ref, o_ref):
        z = zp_ref[0] + zp_ref[1]
        t1 = jnp.maximum(
            lax.dot_general(x_ref[...], u1x_ref[...], _DN,
                            preferred_element_type=jnp.float32)
            + lax.dot_general(z, u1z_ref[...], _DN,
                              preferred_element_type=jnp.float32)
            + c1_ref[...], 0.0)
        t2 = jnp.maximum(
            lax.dot_general(t1, u2_ref[...], _DN,
                            preferred_element_type=jnp.float32) + c2_ref[...],
            0.0)
        o_ref[...] = lax.dot_general(
            t2, u3_ref[...], _DN,
            preferred_element_type=jnp.float32) + c3_ref[...]

    full = lambda shape: pl.BlockSpec(shape, lambda i: (0,) * len(shape))
    return pl.pallas_call(
        body,
        grid=(N // BN,),
        in_specs=[pl.BlockSpec((BN, D), lambda i: (i, 0)),
                  pl.BlockSpec((2, BN, H), lambda i: (0, i, 0)),
                  full((H, D)), full((H, H)), full((1, H)),
                  full((H, H)), full((1, H)),
                  full((OUT, H)), full((1, OUT))],
        out_specs=pl.BlockSpec((BN, OUT), lambda i: (i, 0)),
        out_shape=jax.ShapeDtypeStruct((N, OUT), jnp.float32),
    )(x, zp, U1x, U1z, c1, U2, c2, U3, c3)


def kernel(x, edge_index, edge_attr, W1, b1, W2, b2, W3, b3,
           U1, c1, U2, c2, U3, c3):
    N, D = x.shape
    H = W1.shape[0]
    src = edge_index[0]
    dst = edge_index[1]
    W1x, W1e = W1[:, :D], W1[:, D:]
    U1x, U1z = U1[:, :D], U1[:, D:]

    xw = _xw_matmul(x, W1x)
    xwg = _sc_gather(xw, src)
    m = _edge_mlp(xwg, edge_attr, W1e, b1.reshape(1, -1), W2,
                  b2.reshape(1, -1), W3, b3.reshape(1, -1))
    zp = _sc_scatter_add(m, dst, jnp.zeros((N, H), jnp.float32))
    h = _node_mlp(x, zp, U1x, U1z, c1.reshape(1, -1), U2,
                  c2.reshape(1, -1), U3, c3.reshape(1, -1))
    return h


# speedup vs baseline: 2.3084x; 1.3083x over previous
import functools

import jax
import jax.numpy as jnp
from jax import lax
from jax.experimental import pallas as pl
from jax.experimental.pallas import tpu as pltpu
from jax.experimental.pallas import tpu_sc as plsc

_DN = (((1,), (1,)), ((), ()))


_NBUF = 5


def _sc_gather(table, idx):
    N, H = table.shape
    E = idx.shape[0]
    NC, NS = 2, 16
    NW = NC * NS
    epw = E // NW
    CH = 80
    nch = epw // CH
    assert nch % _NBUF == 0
    mesh = plsc.VectorSubcoreMesh(core_axis_name="c", subcore_axis_name="s")

    @functools.partial(
        pl.kernel,
        out_type=jax.ShapeDtypeStruct((E, H), jnp.float32),
        mesh=mesh,
        compiler_params=pltpu.CompilerParams(use_tc_tiling_on_sc=False),
        scratch_types=[
            pltpu.VMEM((epw,), jnp.int32),
            [pltpu.VMEM((CH, H), jnp.float32) for _ in range(_NBUF)],
            [pltpu.SemaphoreType.DMA for _ in range(_NBUF)],
        ],
    )
    def k(table_hbm, idx_hbm, out_hbm, idx_v, rows_bufs, sems):
        wid = lax.axis_index("s") * NC + lax.axis_index("c")
        base = wid * epw
        pltpu.sync_copy(idx_hbm.at[pl.ds(base, epw)], idx_v)
        for b in range(_NBUF):
            pltpu.async_copy(table_hbm.at[idx_v.at[pl.ds(b * CH, CH)]],
                             rows_bufs[b], sems[b])

        @pl.loop(0, nch, step=_NBUF)
        def _(c0):
            for b in range(_NBUF):
                c = c0 + b
                pltpu.make_async_copy(
                    table_hbm.at[idx_v.at[pl.ds(c * CH, CH)]],
                    rows_bufs[b], sems[b]).wait()
                pltpu.sync_copy(rows_bufs[b],
                                out_hbm.at[pl.ds(base + c * CH, CH)])
                cn = c + _NBUF

                @pl.when(cn < nch)
                def _():
                    pltpu.async_copy(
                        table_hbm.at[idx_v.at[pl.ds(cn * CH, CH)]],
                        rows_bufs[b], sems[b])

    return k(table, idx)


def _edge_mlp(xg, ea, W1x, W1e, b1, W2, b2, W3, b3):
    E, D = xg.shape
    DE = ea.shape[1]
    H = W2.shape[0]
    BE = 2000

    def body(xg_ref, ea_ref, w1x_ref, w1e_ref, b1_ref, w2_ref, b2_ref,
             w3_ref, b3_ref, o_ref):
        h1 = lax.dot_general(
            xg_ref[...], w1x_ref[...], _DN,
            preferred_element_type=jnp.float32) + lax.dot_general(
            ea_ref[...], w1e_ref[...], _DN,
            preferred_element_type=jnp.float32) + b1_ref[...]
        h1 = jnp.maximum(h1, 0.0)
        h2 = jnp.maximum(
            lax.dot_general(h1, w2_ref[...], _DN,
                            preferred_element_type=jnp.float32) + b2_ref[...],
            0.0)
        o_ref[...] = lax.dot_general(
            h2, w3_ref[...], _DN,
            preferred_element_type=jnp.float32) + b3_ref[...]

    full = lambda shape: pl.BlockSpec(shape, lambda i: (0,) * len(shape))
    return pl.pallas_call(
        body,
        grid=(E // BE,),
        in_specs=[pl.BlockSpec((BE, D), lambda i: (i, 0)),
                  pl.BlockSpec((BE, DE), lambda i: (i, 0)),
                  full((H, D)), full((H, DE)), full((1, H)),
                  full((H, H)), full((1, H)),
                  full((H, H)), full((1, H))],
        out_specs=pl.BlockSpec((BE, H), lambda i: (i, 0)),
        out_shape=jax.ShapeDtypeStruct((E, H), jnp.float32),
    )(xg, ea, W1x, W1e, b1, W2, b2, W3, b3)


def _sc_scatter_add(m, dst, zeros):
    E, H = m.shape
    N = zeros.shape[0]
    NC, NS = 2, 16
    NW = NC * NS
    epw = E // NW
    CH = 40
    nch = epw // CH
    rpw = N // NS
    mesh = plsc.VectorSubcoreMesh(core_axis_name="c", subcore_axis_name="s")

    assert nch % _NBUF == 0

    @functools.partial(
        pl.kernel,
        out_type=jax.ShapeDtypeStruct((NC, N, H), jnp.float32),
        mesh=mesh,
        compiler_params=pltpu.CompilerParams(use_tc_tiling_on_sc=False),
        scratch_types=[
            [pltpu.VMEM((CH,), jnp.int32) for _ in range(_NBUF)],
            [pltpu.VMEM((CH, H), jnp.float32) for _ in range(_NBUF)],
            pltpu.VMEM_SHARED((N, H), jnp.float32),
            [pltpu.SemaphoreType.DMA for _ in range(_NBUF)],
            [pltpu.SemaphoreType.DMA for _ in range(_NBUF)],
        ],
    )
    def k(m_hbm, dst_hbm, zeros_hbm, z_hbm, idx_bufs, rows_bufs, acc_sh,
          isems, rsems):
        c = lax.axis_index("c")
        s = lax.axis_index("s")
        wid = s * NC + c
        r0 = s * rpw
        pltpu.sync_copy(zeros_hbm.at[pl.ds(r0, rpw)],
                        acc_sh.at[pl.ds(r0, rpw)])
        plsc.subcore_barrier()

        base = wid * epw
        for b in range(_NBUF):
            off = base + b * CH
            pltpu.async_copy(dst_hbm.at[pl.ds(off, CH)], idx_bufs[b],
                             isems[b])
            pltpu.async_copy(m_hbm.at[pl.ds(off, CH)], rows_bufs[b],
                             rsems[b])

        @pl.loop(0, nch, step=_NBUF)
        def _(c0):
            for b in range(_NBUF):
                ci = c0 + b
                off = base + ci * CH
                pltpu.make_async_copy(dst_hbm.at[pl.ds(off, CH)],
                                      idx_bufs[b], isems[b]).wait()
                pltpu.make_async_copy(m_hbm.at[pl.ds(off, CH)],
                                      rows_bufs[b], rsems[b]).wait()
                pltpu.sync_copy(rows_bufs[b], acc_sh.at[idx_bufs[b]],
                                add=True)
                cn = ci + _NBUF

                @pl.when(cn < nch)
                def _():
                    offn = base + cn * CH
                    pltpu.async_copy(dst_hbm.at[pl.ds(offn, CH)],
                                     idx_bufs[b], isems[b])
                    pltpu.async_copy(m_hbm.at[pl.ds(offn, CH)],
                                     rows_bufs[b], rsems[b])

        plsc.subcore_barrier()
        pltpu.sync_copy(acc_sh.at[pl.ds(r0, rpw)],
                        z_hbm.at[c, pl.ds(r0, rpw)])

    return k(m, dst, zeros)


def _node_mlp(x, zp, U1x, U1z, c1, U2, c2, U3, c3):
    N, D = x.shape
    H = U1z.shape[1]
    OUT = U3.shape[0]
    BN = 2000

    def body(x_ref, zp_ref, u1x_ref, u1z_ref, c1_ref, u2_ref, c2_ref,
             u3_ref, c3_ref, o_ref):
        z = zp_ref[0] + zp_ref[1]
        t1 = jnp.maximum(
            lax.dot_general(x_ref[...], u1x_ref[...], _DN,
                            preferred_element_type=jnp.float32)
            + lax.dot_general(z, u1z_ref[...], _DN,
                              preferred_element_type=jnp.float32)
            + c1_ref[...], 0.0)
        t2 = jnp.maximum(
            lax.dot_general(t1, u2_ref[...], _DN,
                            preferred_element_type=jnp.float32) + c2_ref[...],
            0.0)
        o_ref[...] = lax.dot_general(
            t2, u3_ref[...], _DN,
            preferred_element_type=jnp.float32) + c3_ref[...]

    full = lambda shape: pl.BlockSpec(shape, lambda i: (0,) * len(shape))
    return pl.pallas_call(
        body,
        grid=(N // BN,),
        in_specs=[pl.BlockSpec((BN, D), lambda i: (i, 0)),
                  pl.BlockSpec((2, BN, H), lambda i: (0, i, 0)),
                  full((H, D)), full((H, H)), full((1, H)),
                  full((H, H)), full((1, H)),
                  full((OUT, H)), full((1, OUT))],
        out_specs=pl.BlockSpec((BN, OUT), lambda i: (i, 0)),
        out_shape=jax.ShapeDtypeStruct((N, OUT), jnp.float32),
    )(x, zp, U1x, U1z, c1, U2, c2, U3, c3)


def kernel(x, edge_index, edge_attr, W1, b1, W2, b2, W3, b3,
           U1, c1, U2, c2, U3, c3):
    N, D = x.shape
    H = W1.shape[0]
    src = edge_index[0]
    dst = edge_index[1]
    W1x, W1e = W1[:, :D], W1[:, D:]
    U1x, U1z = U1[:, :D], U1[:, D:]

    xg = _sc_gather(x, src)
    m = _edge_mlp(xg, edge_attr, W1x, W1e, b1.reshape(1, -1), W2,
                  b2.reshape(1, -1), W3, b3.reshape(1, -1))
    zp = _sc_scatter_add(m, dst, jnp.zeros((N, H), jnp.float32))
    h = _node_mlp(x, zp, U1x, U1z, c1.reshape(1, -1), U2,
                  c2.reshape(1, -1), U3, c3.reshape(1, -1))
    return h


# speedup vs baseline: 2.6346x; 1.1413x over previous
import functools

import jax
import jax.numpy as jnp
from jax import lax
from jax.experimental import pallas as pl
from jax.experimental.pallas import tpu as pltpu
from jax.experimental.pallas import tpu_sc as plsc

_DN = (((1,), (1,)), ((), ()))


_NBUF = 5


def _sc_gather(table, idx):
    N, H = table.shape
    E = idx.shape[0]
    NC, NS = 2, 16
    NW = NC * NS
    epw = E // NW
    CH = 80
    nch = epw // CH
    assert nch % _NBUF == 0
    mesh = plsc.VectorSubcoreMesh(core_axis_name="c", subcore_axis_name="s")

    @functools.partial(
        pl.kernel,
        out_type=jax.ShapeDtypeStruct((E, H), jnp.float32),
        mesh=mesh,
        compiler_params=pltpu.CompilerParams(use_tc_tiling_on_sc=False),
        scratch_types=[
            pltpu.VMEM((epw,), jnp.int32),
            [pltpu.VMEM((CH, H), jnp.float32) for _ in range(_NBUF)],
            [pltpu.SemaphoreType.DMA for _ in range(_NBUF)],
        ],
    )
    def k(table_hbm, idx_hbm, out_hbm, idx_v, rows_bufs, sems):
        wid = lax.axis_index("s") * NC + lax.axis_index("c")
        base = wid * epw
        pltpu.sync_copy(idx_hbm.at[pl.ds(base, epw)], idx_v)
        for b in range(_NBUF):
            pltpu.async_copy(table_hbm.at[idx_v.at[pl.ds(b * CH, CH)]],
                             rows_bufs[b], sems[b])

        @pl.loop(0, nch, step=_NBUF)
        def _(c0):
            for b in range(_NBUF):
                c = c0 + b
                pltpu.make_async_copy(
                    table_hbm.at[idx_v.at[pl.ds(c * CH, CH)]],
                    rows_bufs[b], sems[b]).wait()
                pltpu.sync_copy(rows_bufs[b],
                                out_hbm.at[pl.ds(base + c * CH, CH)])
                cn = c + _NBUF

                @pl.when(cn < nch)
                def _():
                    pltpu.async_copy(
                        table_hbm.at[idx_v.at[pl.ds(cn * CH, CH)]],
                        rows_bufs[b], sems[b])

    return k(table, idx)


def _edge_mlp(xg, ea, W1x, W1e, b1, W2, b2, W3, b3):
    E, D = xg.shape
    DE = ea.shape[1]
    H = W2.shape[0]
    BE = 3200

    def body(xg_ref, ea_ref, w1x_ref, w1e_ref, b1_ref, w2_ref, b2_ref,
             w3_ref, b3_ref, o_ref):
        h1 = lax.dot_general(
            xg_ref[...], w1x_ref[...], _DN,
            preferred_element_type=jnp.float32) + lax.dot_general(
            ea_ref[...], w1e_ref[...], _DN,
            preferred_element_type=jnp.float32) + b1_ref[...]
        h1 = jnp.maximum(h1, 0.0)
        h2 = jnp.maximum(
            lax.dot_general(h1, w2_ref[...], _DN,
                            preferred_element_type=jnp.float32) + b2_ref[...],
            0.0)
        m = lax.dot_general(
            h2, w3_ref[...], _DN,
            preferred_element_type=jnp.float32) + b3_ref[...]
        o_ref[:, pl.ds(0, H)] = m

    full = lambda shape: pl.BlockSpec(shape, lambda i: (0,) * len(shape))
    return pl.pallas_call(
        body,
        grid=(E // BE,),
        in_specs=[pl.BlockSpec((BE, D), lambda i: (i, 0)),
                  pl.BlockSpec((BE, DE), lambda i: (i, 0)),
                  full((H, D)), full((H, DE)), full((1, H)),
                  full((H, H)), full((1, H)),
                  full((H, H)), full((1, H))],
        out_specs=pl.BlockSpec((BE, 256), lambda i: (i, 0)),
        out_shape=jax.ShapeDtypeStruct((E, 256), jnp.float32),
    )(xg, ea, W1x, W1e, b1, W2, b2, W3, b3)


def _sc_scatter_add(m, dst, zeros):
    H = zeros.shape[1]
    E = m.shape[0]
    N = zeros.shape[0]
    NC, NS = 2, 16
    NW = NC * NS
    epw = E // NW
    CH = 40
    nch = epw // CH
    rpw = N // NS
    mesh = plsc.VectorSubcoreMesh(core_axis_name="c", subcore_axis_name="s")

    assert nch % _NBUF == 0

    @functools.partial(
        pl.kernel,
        out_type=jax.ShapeDtypeStruct((NC, N, H), jnp.float32),
        mesh=mesh,
        compiler_params=pltpu.CompilerParams(use_tc_tiling_on_sc=False),
        scratch_types=[
            [pltpu.VMEM((CH,), jnp.int32) for _ in range(_NBUF)],
            [pltpu.VMEM((CH, H), jnp.float32) for _ in range(_NBUF)],
            pltpu.VMEM_SHARED((N, H), jnp.float32),
            [pltpu.SemaphoreType.DMA for _ in range(_NBUF)],
            [pltpu.SemaphoreType.DMA for _ in range(_NBUF)],
        ],
    )
    def k(m_hbm, dst_hbm, zeros_hbm, z_hbm, idx_bufs, rows_bufs, acc_sh,
          isems, rsems):
        c = lax.axis_index("c")
        s = lax.axis_index("s")
        wid = s * NC + c
        r0 = s * rpw
        pltpu.sync_copy(zeros_hbm.at[pl.ds(r0, rpw)],
                        acc_sh.at[pl.ds(r0, rpw)])
        plsc.subcore_barrier()

        base = wid * epw

        def load_chunk(off, b):
            pltpu.async_copy(dst_hbm.at[pl.ds(off, CH)], idx_bufs[b],
                             isems[b])
            pltpu.async_copy(m_hbm.at[pl.ds(off, CH), pl.ds(0, H)],
                             rows_bufs[b], rsems[b])

        def wait_chunk(off, b):
            pltpu.make_async_copy(dst_hbm.at[pl.ds(off, CH)],
                                  idx_bufs[b], isems[b]).wait()
            pltpu.make_async_copy(m_hbm.at[pl.ds(off, CH), pl.ds(0, H)],
                                  rows_bufs[b], rsems[b]).wait()

        for b in range(_NBUF):
            load_chunk(base + b * CH, b)

        @pl.loop(0, nch, step=_NBUF)
        def _(c0):
            for b in range(_NBUF):
                ci = c0 + b
                wait_chunk(base + ci * CH, b)
                pltpu.sync_copy(rows_bufs[b],
                                acc_sh.at[idx_bufs[b]], add=True)
                cn = ci + _NBUF

                @pl.when(cn < nch)
                def _():
                    load_chunk(base + cn * CH, b)

        plsc.subcore_barrier()
        pltpu.sync_copy(acc_sh.at[pl.ds(r0, rpw)],
                        z_hbm.at[c, pl.ds(r0, rpw)])

    return k(m, dst, zeros)


def _node_mlp(x, zp, U1x, U1z, c1, U2, c2, U3, c3):
    N, D = x.shape
    H = U1z.shape[1]
    OUT = U3.shape[0]
    BN = 2000

    def body(x_ref, zp_ref, u1x_ref, u1z_ref, c1_ref, u2_ref, c2_ref,
             u3_ref, c3_ref, o_ref):
        z = zp_ref[0] + zp_ref[1]
        t1 = jnp.maximum(
            lax.dot_general(x_ref[...], u1x_ref[...], _DN,
                            preferred_element_type=jnp.float32)
            + lax.dot_general(z, u1z_ref[...], _DN,
                              preferred_element_type=jnp.float32)
            + c1_ref[...], 0.0)
        t2 = jnp.maximum(
            lax.dot_general(t1, u2_ref[...], _DN,
                            preferred_element_type=jnp.float32) + c2_ref[...],
            0.0)
        o_ref[...] = lax.dot_general(
            t2, u3_ref[...], _DN,
            preferred_element_type=jnp.float32) + c3_ref[...]

    full = lambda shape: pl.BlockSpec(shape, lambda i: (0,) * len(shape))
    return pl.pallas_call(
        body,
        grid=(N // BN,),
        in_specs=[pl.BlockSpec((BN, D), lambda i: (i, 0)),
                  pl.BlockSpec((2, BN, H), lambda i: (0, i, 0)),
                  full((H, D)), full((H, H)), full((1, H)),
                  full((H, H)), full((1, H)),
                  full((OUT, H)), full((1, OUT))],
        out_specs=pl.BlockSpec((BN, OUT), lambda i: (i, 0)),
        out_shape=jax.ShapeDtypeStruct((N, OUT), jnp.float32),
    )(x, zp, U1x, U1z, c1, U2, c2, U3, c3)


def kernel(x, edge_index, edge_attr, W1, b1, W2, b2, W3, b3,
           U1, c1, U2, c2, U3, c3):
    N, D = x.shape
    H = W1.shape[0]
    src = edge_index[0]
    dst = edge_index[1]
    W1x, W1e = W1[:, :D], W1[:, D:]
    U1x, U1z = U1[:, :D], U1[:, D:]

    xg = _sc_gather(x, src)
    m = _edge_mlp(xg, edge_attr, W1x, W1e, b1.reshape(1, -1), W2,
                  b2.reshape(1, -1), W3, b3.reshape(1, -1))
    zp = _sc_scatter_add(m, dst, jnp.zeros((N, H), jnp.float32))
    h = _node_mlp(x, zp, U1x, U1z, c1.reshape(1, -1), U2,
                  c2.reshape(1, -1), U3, c3.reshape(1, -1))
    return h


# speedup vs baseline: 3.6456x; 1.3837x over previous
import functools

import jax
import jax.numpy as jnp
from jax import lax
from jax.experimental import pallas as pl
from jax.experimental.pallas import tpu as pltpu
from jax.experimental.pallas import tpu_sc as plsc

_DN = (((1,), (1,)), ((), ()))


_NBUF = 5


def _sc_gather(table, idx):
    N, H = table.shape
    E = idx.shape[0]
    NC, NS = 2, 16
    NW = NC * NS
    epw = E // NW
    CH = 80
    nch = epw // CH
    assert nch % _NBUF == 0
    mesh = plsc.VectorSubcoreMesh(core_axis_name="c", subcore_axis_name="s")

    @functools.partial(
        pl.kernel,
        out_type=jax.ShapeDtypeStruct((E, H), jnp.float32),
        mesh=mesh,
        compiler_params=pltpu.CompilerParams(use_tc_tiling_on_sc=False),
        scratch_types=[
            pltpu.VMEM((epw,), jnp.int32),
            [pltpu.VMEM((CH, H), jnp.float32) for _ in range(_NBUF)],
            [pltpu.SemaphoreType.DMA for _ in range(_NBUF)],
        ],
    )
    def k(table_hbm, idx_hbm, out_hbm, idx_v, rows_bufs, sems):
        wid = lax.axis_index("s") * NC + lax.axis_index("c")
        base = wid * epw
        pltpu.sync_copy(idx_hbm.at[pl.ds(base, epw)], idx_v)
        for b in range(_NBUF):
            pltpu.async_copy(table_hbm.at[idx_v.at[pl.ds(b * CH, CH)]],
                             rows_bufs[b], sems[b])

        @pl.loop(0, nch, step=_NBUF)
        def _(c0):
            for b in range(_NBUF):
                c = c0 + b
                pltpu.make_async_copy(
                    table_hbm.at[idx_v.at[pl.ds(c * CH, CH)]],
                    rows_bufs[b], sems[b]).wait()
                pltpu.sync_copy(rows_bufs[b],
                                out_hbm.at[pl.ds(base + c * CH, CH)])
                cn = c + _NBUF

                @pl.when(cn < nch)
                def _():
                    pltpu.async_copy(
                        table_hbm.at[idx_v.at[pl.ds(cn * CH, CH)]],
                        rows_bufs[b], sems[b])

    return k(table, idx)


def _edge_mlp(xg, ea, W1x, W1e, b1, W2, b2, W3, b3):
    E, D = xg.shape
    DE = ea.shape[1]
    H = W2.shape[0]
    BE = 3200

    def body(xg_ref, ea_ref, w1x_ref, w1e_ref, b1_ref, w2_ref, b2_ref,
             w3_ref, b3_ref, oa_ref, ob_ref):
        h1 = lax.dot_general(
            xg_ref[...], w1x_ref[...], _DN,
            preferred_element_type=jnp.float32) + lax.dot_general(
            ea_ref[...], w1e_ref[...], _DN,
            preferred_element_type=jnp.float32) + b1_ref[...]
        h1 = jnp.maximum(h1, 0.0)
        h2 = jnp.maximum(
            lax.dot_general(h1, w2_ref[...], _DN,
                            preferred_element_type=jnp.float32) + b2_ref[...],
            0.0)
        m = lax.dot_general(
            h2, w3_ref[...], _DN,
            preferred_element_type=jnp.float32) + b3_ref[...]
        oa_ref[...] = m[:, :D]
        ob_ref[:, pl.ds(0, H - D)] = m[:, D:]

    full = lambda shape: pl.BlockSpec(shape, lambda i: (0,) * len(shape))
    return pl.pallas_call(
        body,
        grid=(E // BE,),
        in_specs=[pl.BlockSpec((BE, D), lambda i: (i, 0)),
                  pl.BlockSpec((BE, DE), lambda i: (i, 0)),
                  full((H, D)), full((H, DE)), full((1, H)),
                  full((H, H)), full((1, H)),
                  full((H, H)), full((1, H))],
        out_specs=[pl.BlockSpec((BE, D), lambda i: (i, 0)),
                   pl.BlockSpec((BE, D), lambda i: (i, 0))],
        out_shape=[jax.ShapeDtypeStruct((E, D), jnp.float32),
                   jax.ShapeDtypeStruct((E, D), jnp.float32)],
    )(xg, ea, W1x, W1e, b1, W2, b2, W3, b3)


def _sc_scatter_add(ma, mb, dst, zeros):
    E, D = ma.shape
    HB = 16
    N = zeros.shape[0]
    NC, NS = 2, 16
    NW = NC * NS
    epw = E // NW
    CH = 40
    nch = epw // CH
    rpw = N // NS
    mesh = plsc.VectorSubcoreMesh(core_axis_name="c", subcore_axis_name="s")

    assert nch % _NBUF == 0

    @functools.partial(
        pl.kernel,
        out_type=(jax.ShapeDtypeStruct((NC, N, D), jnp.float32),
                  jax.ShapeDtypeStruct((NC, N, D), jnp.float32)),
        mesh=mesh,
        compiler_params=pltpu.CompilerParams(use_tc_tiling_on_sc=False),
        scratch_types=[
            [pltpu.VMEM((CH,), jnp.int32) for _ in range(_NBUF)],
            [pltpu.VMEM((CH, D), jnp.float32) for _ in range(_NBUF)],
            [pltpu.VMEM((CH, HB), jnp.float32) for _ in range(_NBUF)],
            pltpu.VMEM_SHARED((N, D), jnp.float32),
            pltpu.VMEM_SHARED((N, HB), jnp.float32),
            [pltpu.SemaphoreType.DMA for _ in range(_NBUF)],
            [pltpu.SemaphoreType.DMA for _ in range(_NBUF)],
            [pltpu.SemaphoreType.DMA for _ in range(_NBUF)],
        ],
    )
    def k(ma_hbm, mb_hbm, dst_hbm, zeros_hbm, za_hbm, zb_hbm,
          idx_bufs, ra_bufs, rb_bufs, acca_sh, accb_sh, isems, asems, bsems):
        c = lax.axis_index("c")
        s = lax.axis_index("s")
        wid = s * NC + c
        r0 = s * rpw
        pltpu.sync_copy(zeros_hbm.at[pl.ds(r0, rpw)],
                        acca_sh.at[pl.ds(r0, rpw)])
        pltpu.sync_copy(zeros_hbm.at[pl.ds(r0, rpw), pl.ds(0, HB)],
                        accb_sh.at[pl.ds(r0, rpw)])
        plsc.subcore_barrier()

        base = wid * epw

        def load_chunk(off, b):
            pltpu.async_copy(dst_hbm.at[pl.ds(off, CH)], idx_bufs[b],
                             isems[b])
            pltpu.async_copy(ma_hbm.at[pl.ds(off, CH)], ra_bufs[b], asems[b])
            pltpu.async_copy(mb_hbm.at[pl.ds(off, CH), pl.ds(0, HB)],
                             rb_bufs[b], bsems[b])

        def wait_chunk(off, b):
            pltpu.make_async_copy(dst_hbm.at[pl.ds(off, CH)],
                                  idx_bufs[b], isems[b]).wait()
            pltpu.make_async_copy(ma_hbm.at[pl.ds(off, CH)],
                                  ra_bufs[b], asems[b]).wait()
            pltpu.make_async_copy(mb_hbm.at[pl.ds(off, CH), pl.ds(0, HB)],
                                  rb_bufs[b], bsems[b]).wait()

        for b in range(_NBUF):
            load_chunk(base + b * CH, b)

        @pl.loop(0, nch, step=_NBUF)
        def _(c0):
            for b in range(_NBUF):
                ci = c0 + b
                wait_chunk(base + ci * CH, b)
                pltpu.sync_copy(ra_bufs[b],
                                acca_sh.at[idx_bufs[b]], add=True)
                pltpu.sync_copy(rb_bufs[b],
                                accb_sh.at[idx_bufs[b]], add=True)
                cn = ci + _NBUF

                @pl.when(cn < nch)
                def _():
                    load_chunk(base + cn * CH, b)

        plsc.subcore_barrier()
        pltpu.sync_copy(acca_sh.at[pl.ds(r0, rpw)],
                        za_hbm.at[c, pl.ds(r0, rpw)])
        pltpu.sync_copy(accb_sh.at[pl.ds(r0, rpw)],
                        zb_hbm.at[c, pl.ds(r0, rpw), pl.ds(0, HB)])

    return k(ma, mb, dst, zeros)


def _node_mlp(x, za, zb, U1x, U1za, U1zb, c1, U2, c2, U3, c3):
    N, D = x.shape
    H = U2.shape[0]
    HB = U1zb.shape[1]
    OUT = U3.shape[0]
    BN = 2000

    def body(x_ref, za_ref, zb_ref, u1x_ref, u1za_ref, u1zb_ref, c1_ref,
             u2_ref, c2_ref, u3_ref, c3_ref, o_ref):
        za = za_ref[0] + za_ref[1]
        zb = (zb_ref[0] + zb_ref[1])[:, :HB]
        t1 = jnp.maximum(
            lax.dot_general(x_ref[...], u1x_ref[...], _DN,
                            preferred_element_type=jnp.float32)
            + lax.dot_general(za, u1za_ref[...], _DN,
                              preferred_element_type=jnp.float32)
            + lax.dot_general(zb, u1zb_ref[...], _DN,
                              preferred_element_type=jnp.float32)
            + c1_ref[...], 0.0)
        t2 = jnp.maximum(
            lax.dot_general(t1, u2_ref[...], _DN,
                            preferred_element_type=jnp.float32) + c2_ref[...],
            0.0)
        o_ref[...] = lax.dot_general(
            t2, u3_ref[...], _DN,
            preferred_element_type=jnp.float32) + c3_ref[...]

    full = lambda shape: pl.BlockSpec(shape, lambda i: (0,) * len(shape))
    return pl.pallas_call(
        body,
        grid=(N // BN,),
        in_specs=[pl.BlockSpec((BN, D), lambda i: (i, 0)),
                  pl.BlockSpec((2, BN, D), lambda i: (0, i, 0)),
                  pl.BlockSpec((2, BN, D), lambda i: (0, i, 0)),
                  full((H, D)), full((H, D)), full((H, HB)), full((1, H)),
                  full((H, H)), full((1, H)),
                  full((OUT, H)), full((1, OUT))],
        out_specs=pl.BlockSpec((BN, OUT), lambda i: (i, 0)),
        out_shape=jax.ShapeDtypeStruct((N, OUT), jnp.float32),
    )(x, za, zb, U1x, U1za, U1zb, c1, U2, c2, U3, c3)


def kernel(x, edge_index, edge_attr, W1, b1, W2, b2, W3, b3,
           U1, c1, U2, c2, U3, c3):
    N, D = x.shape
    src = edge_index[0]
    dst = edge_index[1]
    W1x, W1e = W1[:, :D], W1[:, D:]
    U1x, U1za, U1zb = U1[:, :D], U1[:, D:2 * D], U1[:, 2 * D:]

    xg = _sc_gather(x, src)
    ma, mb = _edge_mlp(xg, edge_attr, W1x, W1e, b1.reshape(1, -1), W2,
                       b2.reshape(1, -1), W3, b3.reshape(1, -1))
    za, zb = _sc_scatter_add(ma, mb, dst, jnp.zeros((N, D), jnp.float32))
    h = _node_mlp(x, za, zb, U1x, U1za, U1zb, c1.reshape(1, -1), U2,
                  c2.reshape(1, -1), U3, c3.reshape(1, -1))
    return h


# speedup vs baseline: 4.6130x; 1.2654x over previous
import functools

import jax
import jax.numpy as jnp
from jax import lax
from jax.experimental import pallas as pl
from jax.experimental.pallas import tpu as pltpu
from jax.experimental.pallas import tpu_sc as plsc

_DN = (((1,), (1,)), ((), ()))


_NBUF = 5


def _sc_gather(table, idx, CH):
    N, H = table.shape
    E = idx.shape[0]
    NC, NS = 2, 16
    NW = NC * NS
    epw = E // NW
    nch = epw // CH
    assert nch % _NBUF == 0
    mesh = plsc.VectorSubcoreMesh(core_axis_name="c", subcore_axis_name="s")

    @functools.partial(
        pl.kernel,
        out_type=jax.ShapeDtypeStruct((E, H), jnp.float32),
        mesh=mesh,
        compiler_params=pltpu.CompilerParams(use_tc_tiling_on_sc=True),
        scratch_types=[
            pltpu.VMEM((epw,), jnp.int32),
            [pltpu.VMEM((CH, H), jnp.float32) for _ in range(_NBUF)],
            [pltpu.SemaphoreType.DMA for _ in range(_NBUF)],
        ],
    )
    def k(table_hbm, idx_hbm, out_hbm, idx_v, rows_bufs, sems):
        wid = lax.axis_index("s") * NC + lax.axis_index("c")
        base = wid * epw
        pltpu.sync_copy(idx_hbm.at[pl.ds(base, epw)], idx_v)
        for b in range(_NBUF):
            pltpu.async_copy(table_hbm.at[idx_v.at[pl.ds(b * CH, CH)]],
                             rows_bufs[b], sems[b])

        @pl.loop(0, nch, step=_NBUF)
        def _(c0):
            for b in range(_NBUF):
                c = c0 + b
                pltpu.make_async_copy(
                    table_hbm.at[idx_v.at[pl.ds(c * CH, CH)]],
                    rows_bufs[b], sems[b]).wait()
                pltpu.sync_copy(rows_bufs[b],
                                out_hbm.at[pl.ds(base + c * CH, CH)])
                cn = c + _NBUF

                @pl.when(cn < nch)
                def _():
                    pltpu.async_copy(
                        table_hbm.at[idx_v.at[pl.ds(cn * CH, CH)]],
                        rows_bufs[b], sems[b])

    return k(table, idx)


def _edge_mlp(xg, ea_t, W1x, W1e, b1, W2, b2, W3, b3, ea_row_off=0):
    E, D = xg.shape
    DE = ea_t.shape[0]
    H = W2.shape[0]
    BE = 3200
    eb = ea_row_off // BE

    def body(xg_ref, ea_ref, w1x_ref, w1e_ref, b1_ref, w2_ref, b2_ref,
             w3_ref, b3_ref, oa_ref, ob_ref):
        h1 = lax.dot_general(
            xg_ref[...], w1x_ref[...], _DN,
            preferred_element_type=jnp.float32) + lax.dot_general(
            ea_ref[...], w1e_ref[...], (((0,), (1,)), ((), ())),
            preferred_element_type=jnp.float32) + b1_ref[...]
        h1 = jnp.maximum(h1, 0.0)
        h2 = jnp.maximum(
            lax.dot_general(h1, w2_ref[...], _DN,
                            preferred_element_type=jnp.float32) + b2_ref[...],
            0.0)
        m = lax.dot_general(
            h2, w3_ref[...], _DN,
            preferred_element_type=jnp.float32) + b3_ref[...]
        oa_ref[...] = m[:, :D]
        ob_ref[:, pl.ds(0, H - D)] = m[:, D:]

    full = lambda shape: pl.BlockSpec(shape, lambda i: (0,) * len(shape))
    return pl.pallas_call(
        body,
        grid=(E // BE,),
        in_specs=[pl.BlockSpec((BE, D), lambda i: (i, 0)),
                  pl.BlockSpec((DE, BE), lambda i: (0, i + eb)),
                  full((H, D)), full((H, DE)), full((1, H)),
                  full((H, H)), full((1, H)),
                  full((H, H)), full((1, H))],
        out_specs=[pl.BlockSpec((BE, D), lambda i: (i, 0)),
                   pl.BlockSpec((BE, D), lambda i: (i, 0))],
        out_shape=[jax.ShapeDtypeStruct((E, D), jnp.float32),
                   jax.ShapeDtypeStruct((E, D), jnp.float32)],
    )(xg, ea_t, W1x, W1e, b1, W2, b2, W3, b3)


def _sc_scatter_add(ma, mb, dst, zeros, CH):
    E, D = ma.shape
    HB = 16
    N = zeros.shape[0]
    NC, NS = 2, 16
    NW = NC * NS
    epw = E // NW
    nch = epw // CH
    rpw = N // NS
    mesh = plsc.VectorSubcoreMesh(core_axis_name="c", subcore_axis_name="s")

    assert nch % _NBUF == 0

    @functools.partial(
        pl.kernel,
        out_type=(jax.ShapeDtypeStruct((NC, N, D), jnp.float32),
                  jax.ShapeDtypeStruct((NC, N, D), jnp.float32)),
        mesh=mesh,
        compiler_params=pltpu.CompilerParams(use_tc_tiling_on_sc=False),
        scratch_types=[
            [pltpu.VMEM((CH,), jnp.int32) for _ in range(_NBUF)],
            [pltpu.VMEM((CH, D), jnp.float32) for _ in range(_NBUF)],
            [pltpu.VMEM((CH, HB), jnp.float32) for _ in range(_NBUF)],
            pltpu.VMEM_SHARED((N, D), jnp.float32),
            pltpu.VMEM_SHARED((N, HB), jnp.float32),
            [pltpu.SemaphoreType.DMA for _ in range(_NBUF)],
            [pltpu.SemaphoreType.DMA for _ in range(_NBUF)],
            [pltpu.SemaphoreType.DMA for _ in range(_NBUF)],
        ],
    )
    def k(ma_hbm, mb_hbm, dst_hbm, zeros_hbm, za_hbm, zb_hbm,
          idx_bufs, ra_bufs, rb_bufs, acca_sh, accb_sh, isems, asems, bsems):
        c = lax.axis_index("c")
        s = lax.axis_index("s")
        wid = s * NC + c
        r0 = s * rpw
        pltpu.sync_copy(zeros_hbm.at[pl.ds(r0, rpw)],
                        acca_sh.at[pl.ds(r0, rpw)])
        pltpu.sync_copy(zeros_hbm.at[pl.ds(r0, rpw), pl.ds(0, HB)],
                        accb_sh.at[pl.ds(r0, rpw)])
        plsc.subcore_barrier()

        base = wid * epw

        def load_chunk(off, b):
            pltpu.async_copy(dst_hbm.at[pl.ds(off, CH)], idx_bufs[b],
                             isems[b])
            pltpu.async_copy(ma_hbm.at[pl.ds(off, CH)], ra_bufs[b], asems[b])
            pltpu.async_copy(mb_hbm.at[pl.ds(off, CH), pl.ds(0, HB)],
                             rb_bufs[b], bsems[b])

        def wait_chunk(off, b):
            pltpu.make_async_copy(dst_hbm.at[pl.ds(off, CH)],
                                  idx_bufs[b], isems[b]).wait()
            pltpu.make_async_copy(ma_hbm.at[pl.ds(off, CH)],
                                  ra_bufs[b], asems[b]).wait()
            pltpu.make_async_copy(mb_hbm.at[pl.ds(off, CH), pl.ds(0, HB)],
                                  rb_bufs[b], bsems[b]).wait()

        for b in range(_NBUF):
            load_chunk(base + b * CH, b)

        @pl.loop(0, nch, step=_NBUF)
        def _(c0):
            for b in range(_NBUF):
                ci = c0 + b
                wait_chunk(base + ci * CH, b)
                pltpu.sync_copy(ra_bufs[b],
                                acca_sh.at[idx_bufs[b]], add=True)
                pltpu.sync_copy(rb_bufs[b],
                                accb_sh.at[idx_bufs[b]], add=True)
                cn = ci + _NBUF

                @pl.when(cn < nch)
                def _():
                    load_chunk(base + cn * CH, b)

        plsc.subcore_barrier()
        pltpu.sync_copy(acca_sh.at[pl.ds(r0, rpw)],
                        za_hbm.at[c, pl.ds(r0, rpw)])
        pltpu.sync_copy(accb_sh.at[pl.ds(r0, rpw)],
                        zb_hbm.at[c, pl.ds(r0, rpw), pl.ds(0, HB)])

    return k(ma, mb, dst, zeros)


def _node_mlp(x, zas, zbs, U1x, U1za, U1zb, c1, U2, c2, U3, c3):
    N, D = x.shape
    H = U2.shape[0]
    HB = U1zb.shape[1]
    OUT = U3.shape[0]
    BN = 2000
    S = len(zas)

    def body(*refs):
        x_ref = refs[0]
        za_refs = refs[1:1 + S]
        zb_refs = refs[1 + S:1 + 2 * S]
        (u1x_ref, u1za_ref, u1zb_ref, c1_ref, u2_ref, c2_ref, u3_ref,
         c3_ref, o_ref) = refs[1 + 2 * S:]
        za = sum(r[0] + r[1] for r in za_refs)
        zb = sum(r[0] + r[1] for r in zb_refs)[:, :HB]
        t1 = jnp.maximum(
            lax.dot_general(x_ref[...], u1x_ref[...], _DN,
                            preferred_element_type=jnp.float32)
            + lax.dot_general(za, u1za_ref[...], _DN,
                              preferred_element_type=jnp.float32)
            + lax.dot_general(zb, u1zb_ref[...], _DN,
                              preferred_element_type=jnp.float32)
            + c1_ref[...], 0.0)
        t2 = jnp.maximum(
            lax.dot_general(t1, u2_ref[...], _DN,
                            preferred_element_type=jnp.float32) + c2_ref[...],
            0.0)
        o_ref[...] = lax.dot_general(
            t2, u3_ref[...], _DN,
            preferred_element_type=jnp.float32) + c3_ref[...]

    full = lambda shape: pl.BlockSpec(shape, lambda i: (0,) * len(shape))
    zspec = pl.BlockSpec((2, BN, D), lambda i: (0, i, 0))
    return pl.pallas_call(
        body,
        grid=(N // BN,),
        in_specs=([pl.BlockSpec((BN, D), lambda i: (i, 0))]
                  + [zspec] * (2 * S)
                  + [full((H, D)), full((H, D)), full((H, HB)),
                     full((1, H)), full((H, H)), full((1, H)),
                     full((OUT, H)), full((1, OUT))]),
        out_specs=pl.BlockSpec((BN, OUT), lambda i: (i, 0)),
        out_shape=jax.ShapeDtypeStruct((N, OUT), jnp.float32),
    )(x, *zas, *zbs, U1x, U1za, U1zb, c1, U2, c2, U3, c3)


def kernel(x, edge_index, edge_attr, W1, b1, W2, b2, W3, b3,
           U1, c1, U2, c2, U3, c3):
    N, D = x.shape
    E = edge_index.shape[1]
    src = edge_index[0]
    dst = edge_index[1]
    W1x, W1e = W1[:, :D], W1[:, D:]
    U1x, U1za, U1zb = U1[:, :D], U1[:, D:2 * D], U1[:, 2 * D:]
    b1r, b2r, b3r = b1.reshape(1, -1), b2.reshape(1, -1), b3.reshape(1, -1)
    zeros = jnp.zeros((N, D), jnp.float32)
    ea_t = edge_attr.T

    S = 2
    E2 = E // S
    zas, zbs = [], []
    for s in range(S):
        sl = slice(s * E2, (s + 1) * E2)
        xg = _sc_gather(x, src[sl], CH=40)
        ma, mb = _edge_mlp(xg, ea_t, W1x, W1e, b1r, W2, b2r,
                           W3, b3r, ea_row_off=s * E2)
        za, zb = _sc_scatter_add(ma, mb, dst[sl], zeros, CH=40)
        zas.append(za)
        zbs.append(zb)
    h = _node_mlp(x, zas, zbs, U1x, U1za, U1zb, c1.reshape(1, -1), U2,
                  c2.reshape(1, -1), U3, c3.reshape(1, -1))
    return h
